# Initial kernel scaffold; baseline (speedup 1.0000x reference)
import jax
import jax.numpy as jnp
from jax.experimental import pallas as pl


def kernel(inputs, embeddings):
    raise NotImplementedError("write your pallas kernel here")


# speedup vs baseline: 1.3387x; 1.3387x over previous
import functools

import jax
import jax.numpy as jnp
from jax import lax
from jax.experimental import pallas as pl
from jax.experimental.pallas import tpu as pltpu
from jax.experimental.pallas import tpu_sc as plsc

M = 16384
D = 256
N = 8192

TILE_M = 512
M_TILES = M // TILE_M
CHUNKS_N = (0, 2816, 5632, 8192)


def _dist_argmin_body(x_ref, e_ref, idx_ref, mv_ref):
    x = x_ref[...]
    xsq = jnp.sum(x * x, axis=1, keepdims=True)
    best_v = None
    best_i = None
    for c in range(3):
        lo, hi = CHUNKS_N[c], CHUNKS_N[c + 1]
        e = e_ref[:, lo:hi]
        dot = jnp.dot(x, e, preferred_element_type=jnp.float32)
        esq = jnp.sum(e * e, axis=0, keepdims=True)
        d = xsq - 2.0 * dot + esq
        lmin = jnp.min(d, axis=1)
        cols = lax.broadcasted_iota(jnp.int32, d.shape, 1)
        cand = jnp.where(d == lmin[:, None], cols, jnp.int32(N))
        lidx = jnp.min(cand, axis=1) + jnp.int32(lo)
        if c == 0:
            best_v, best_i = lmin, lidx
        else:
            bq = best_v.astype(jnp.bfloat16).astype(jnp.float32)
            take = lmin < bq
            best_v = jnp.where(take, lmin, best_v)
            best_i = jnp.where(take, lidx, best_i)
    idx_ref[0, 0, :] = best_i
    mv_ref[0, 0, :] = best_v


def _dist_argmin(flat_x, embeddings):
    return pl.pallas_call(
        _dist_argmin_body,
        grid=(M_TILES,),
        in_specs=[
            pl.BlockSpec((TILE_M, D), lambda m: (m, 0)),
            pl.BlockSpec((D, N), lambda m: (0, 0)),
        ],
        out_specs=[
            pl.BlockSpec((1, 1, TILE_M), lambda m: (m, 0, 0)),
            pl.BlockSpec((1, 1, TILE_M), lambda m: (m, 0, 0)),
        ],
        out_shape=[
            jax.ShapeDtypeStruct((M_TILES, 1, TILE_M), jnp.int32),
            jax.ShapeDtypeStruct((M_TILES, 1, TILE_M), jnp.float32),
        ],
    )(flat_x, embeddings)


_NC = 2
_NW = _NC * 16
B_PER_W = M // _NW
CH = 128
CHUNKS = B_PER_W // CH


@functools.cache
def _make_sc_gather():
    mesh = plsc.VectorSubcoreMesh(core_axis_name="c", subcore_axis_name="s")

    @functools.partial(
        pl.kernel,
        mesh=mesh,
        out_type=jax.ShapeDtypeStruct((M, D), jnp.float32),
        scratch_types=[
            pltpu.VMEM((CH,), jnp.int32),
            pltpu.VMEM((CH, D), jnp.float32),
            pltpu.SemaphoreType.DMA,
        ],
    )
    def _sc_gather(table_hbm, idx_hbm, out_hbm, idx_v, rows_v, sem):
        wid = lax.axis_index("s") * _NC + lax.axis_index("c")
        base = wid * B_PER_W
        for c in range(CHUNKS):
            off = base + c * CH
            pltpu.sync_copy(idx_hbm.at[pl.ds(off, CH)], idx_v)
            pltpu.async_copy(table_hbm.at[idx_v], rows_v, sem).wait()
            pltpu.sync_copy(rows_v, out_hbm.at[pl.ds(off, CH)])

    return _sc_gather


def kernel(inputs, embeddings):
    flat_x = inputs.reshape(M, D)
    idx3, mv3 = _dist_argmin(flat_x, embeddings)
    idx = idx3.reshape(M)
    table = embeddings.T
    quantized = _make_sc_gather()(table, idx)
    quantized = quantized.reshape(inputs.shape)
    e_latent_loss = jnp.sum(mv3) / jnp.float32(M * D)
    encoding_indices = idx.reshape(inputs.shape[:-1])
    return (quantized, e_latent_loss, encoding_indices)


# speedup vs baseline: 1.4482x; 1.0818x over previous
import functools

import jax
import jax.numpy as jnp
from jax import lax
from jax.experimental import pallas as pl
from jax.experimental.pallas import tpu as pltpu
from jax.experimental.pallas import tpu_sc as plsc

M = 16384
D = 256
N = 8192

TILE_M = 1024
M_TILES = M // TILE_M
CHUNKS_N = (0, 2816, 5632, 8192)


def _dist_argmin_body(x_ref, e_ref, idx_ref, mv_ref):
    x = x_ref[...]
    xsq = jnp.sum(x * x, axis=1, keepdims=True)
    xb = x.astype(jnp.bfloat16)
    best_v = None
    best_i = None
    for c in range(3):
        lo, hi = CHUNKS_N[c], CHUNKS_N[c + 1]
        e = e_ref[:, lo:hi]
        dot = jnp.dot(xb, e.astype(jnp.bfloat16),
                      preferred_element_type=jnp.float32)
        esq = jnp.sum(e * e, axis=0, keepdims=True)
        d = xsq - 2.0 * dot + esq
        lmin = jnp.min(d, axis=1)
        cols = lax.broadcasted_iota(jnp.int32, d.shape, 1)
        cand = jnp.where(d == lmin[:, None], cols, jnp.int32(N))
        lidx = jnp.min(cand, axis=1) + jnp.int32(lo)
        if c == 0:
            best_v, best_i = lmin, lidx
        else:
            bq = best_v.astype(jnp.bfloat16).astype(jnp.float32)
            take = lmin < bq
            best_v = jnp.where(take, lmin, best_v)
            best_i = jnp.where(take, lidx, best_i)
    idx_ref[0, 0, :] = best_i
    mv_ref[0, 0, :] = best_v


def _dist_argmin(flat_x, embeddings):
    return pl.pallas_call(
        _dist_argmin_body,
        grid=(M_TILES,),
        in_specs=[
            pl.BlockSpec((TILE_M, D), lambda m: (m, 0)),
            pl.BlockSpec((D, N), lambda m: (0, 0)),
        ],
        out_specs=[
            pl.BlockSpec((1, 1, TILE_M), lambda m: (m, 0, 0)),
            pl.BlockSpec((1, 1, TILE_M), lambda m: (m, 0, 0)),
        ],
        out_shape=[
            jax.ShapeDtypeStruct((M_TILES, 1, TILE_M), jnp.int32),
            jax.ShapeDtypeStruct((M_TILES, 1, TILE_M), jnp.float32),
        ],
    )(flat_x, embeddings)


_NC = 2
_NW = _NC * 16
B_PER_W = M // _NW
CH = 128
CHUNKS = B_PER_W // CH


@functools.cache
def _make_sc_gather():
    mesh = plsc.VectorSubcoreMesh(core_axis_name="c", subcore_axis_name="s")

    @functools.partial(
        pl.kernel,
        mesh=mesh,
        out_type=jax.ShapeDtypeStruct((M, D), jnp.float32),
        scratch_types=[
            pltpu.VMEM((CH,), jnp.int32),
            pltpu.VMEM((CH, D), jnp.float32),
            pltpu.SemaphoreType.DMA,
        ],
    )
    def _sc_gather(table_hbm, idx_hbm, out_hbm, idx_v, rows_v, sem):
        wid = lax.axis_index("s") * _NC + lax.axis_index("c")
        base = wid * B_PER_W
        for c in range(CHUNKS):
            off = base + c * CH
            pltpu.sync_copy(idx_hbm.at[pl.ds(off, CH)], idx_v)
            pltpu.async_copy(table_hbm.at[idx_v], rows_v, sem).wait()
            pltpu.sync_copy(rows_v, out_hbm.at[pl.ds(off, CH)])

    return _sc_gather


def kernel(inputs, embeddings):
    flat_x = inputs.reshape(M, D)
    idx3, mv3 = _dist_argmin(flat_x, embeddings)
    idx = idx3.reshape(M)
    table = embeddings.T
    quantized = _make_sc_gather()(table, idx)
    quantized = quantized.reshape(inputs.shape)
    e_latent_loss = jnp.sum(mv3) / jnp.float32(M * D)
    encoding_indices = idx.reshape(inputs.shape[:-1])
    return (quantized, e_latent_loss, encoding_indices)
